# Initial kernel scaffold; baseline (speedup 1.0000x reference)
import jax
import jax.numpy as jnp
from jax.experimental import pallas as pl


def kernel(sub_x, sub_edge_index, vnr_x, vnr_edge_index, Ws1, bs1, Ws2, bs2, Ws3, bs3, Wv1, bv1, Wv2, bv2, Wv3, bv3, Wa, Wntn, Vntn, bntn, Wfc, bfc, Wp, bp, Wval, bval):
    raise NotImplementedError("write your pallas kernel here")


# speedup vs baseline: 12.2941x; 12.2941x over previous
import functools

import jax
import jax.numpy as jnp
from jax import lax
from jax.experimental import pallas as pl
from jax.experimental.pallas import tpu as pltpu
from jax.experimental.pallas import tpu_sc as plsc

F32 = jnp.float32

N_SUB = 10000
N_PAD = 10112
E_SUB = 320000
NC, NS = 2, 16
NW = NC * NS
CHUNK = 128
K_CHUNKS = -(-E_SUB // (NW * CHUNK))
E_PAD = NW * K_CHUNKS * CHUNK
ROWS_PER_TILE = N_PAD // NS
DEG_W = 16


def _sc_mesh():
    return plsc.VectorSubcoreMesh(
        core_axis_name="c", subcore_axis_name="s",
        num_cores=NC, num_subcores=NS)


def _deg_body(ones_hbm, dst_hbm, part_hbm, acc, ones_buf, idx_buf):
    cid = lax.axis_index("c")
    sid = lax.axis_index("s")
    wid = cid * NS + sid
    r0 = sid * ROWS_PER_TILE
    pltpu.sync_copy(ones_hbm.at[pl.ds(r0, ROWS_PER_TILE)],
                    acc.at[pl.ds(r0, ROWS_PER_TILE)])
    pltpu.sync_copy(ones_hbm.at[pl.ds(0, CHUNK)], ones_buf)
    plsc.subcore_barrier()

    def step(k, carry):
        pltpu.sync_copy(dst_hbm.at[wid, k], idx_buf)
        pltpu.sync_copy(ones_buf, acc.at[idx_buf], add=True)
        return carry

    lax.fori_loop(0, K_CHUNKS, step, 0)
    plsc.subcore_barrier()
    pltpu.sync_copy(acc.at[pl.ds(r0, ROWS_PER_TILE)],
                    part_hbm.at[cid, pl.ds(r0, ROWS_PER_TILE)])


_sc_fns = {}


def _deg_call(ones, dst):
    if "deg" not in _sc_fns:
        _sc_fns["deg"] = functools.partial(
            pl.kernel,
            out_type=jax.ShapeDtypeStruct((NC, N_PAD, DEG_W), F32),
            mesh=_sc_mesh(),
            scratch_types=[
                pltpu.VMEM_SHARED((N_PAD, DEG_W), F32),
                pltpu.VMEM((CHUNK, DEG_W), F32),
                pltpu.VMEM((CHUNK,), jnp.int32),
            ],
            compiler_params=pltpu.CompilerParams(use_tc_tiling_on_sc=False),
        )(_deg_body)
    return _sc_fns["deg"](ones, dst)


def _make_agg(feat):
    def body(hs_hbm, src_hbm, dst_hbm, part_hbm, acc, rows, sidx, didx, sem):
        cid = lax.axis_index("c")
        sid = lax.axis_index("s")
        wid = cid * NS + sid
        r0 = sid * ROWS_PER_TILE
        pltpu.sync_copy(hs_hbm.at[pl.ds(r0, ROWS_PER_TILE)],
                        acc.at[pl.ds(r0, ROWS_PER_TILE)])
        plsc.subcore_barrier()

        def step(k, carry):
            pltpu.sync_copy(src_hbm.at[wid, k], sidx)
            pltpu.sync_copy(dst_hbm.at[wid, k], didx)
            pltpu.async_copy(hs_hbm.at[sidx], rows, sem).wait()
            pltpu.sync_copy(rows, acc.at[didx], add=True)
            return carry

        lax.fori_loop(0, K_CHUNKS, step, 0)
        plsc.subcore_barrier()
        pltpu.sync_copy(acc.at[pl.ds(r0, ROWS_PER_TILE)],
                        part_hbm.at[cid, pl.ds(r0, ROWS_PER_TILE)])

    def call(hs, src, dst):
        key = ("agg", feat)
        if key not in _sc_fns:
            _sc_fns[key] = functools.partial(
                pl.kernel,
                out_type=jax.ShapeDtypeStruct((NC, N_PAD, feat), F32),
                mesh=_sc_mesh(),
                scratch_types=[
                    pltpu.VMEM_SHARED((N_PAD, feat), F32),
                    pltpu.VMEM((CHUNK, feat), F32),
                    pltpu.VMEM((CHUNK,), jnp.int32),
                    pltpu.VMEM((CHUNK,), jnp.int32),
                    pltpu.SemaphoreType.DMA,
                ],
                compiler_params=pltpu.CompilerParams(
                    use_tc_tiling_on_sc=False),
            )(body)
        return _sc_fns[key](hs, src, dst)

    return call


_agg_64 = _make_agg(64)
_agg_32 = _make_agg(32)
_agg_16 = _make_agg(16)


def _tc1_body(p_ref, x_ref, w_ref, b_ref, hs_ref, inv_ref):
    p = p_ref[...]
    deg = p[0, :, 0:1] + p[1, :, 0:1] - 1.0
    inv = lax.rsqrt(deg)
    h = jnp.dot(x_ref[...], w_ref[...], preferred_element_type=F32, precision=jax.lax.Precision.HIGHEST) + b_ref[...]
    hs_ref[...] = h * inv
    inv_ref[...] = inv


def _tc1(parts, x, w, b):
    return pl.pallas_call(
        _tc1_body,
        out_shape=[jax.ShapeDtypeStruct((N_PAD, w.shape[1]), F32),
                   jax.ShapeDtypeStruct((N_PAD, 1), F32)],
    )(parts, x, w, b)


def _tcmid_body(p_ref, hs_ref, inv_ref, w_ref, b_ref, out_ref):
    p = p_ref[...]
    inv = inv_ref[...]
    a = jnp.maximum((p[0] + p[1] - hs_ref[...]) * inv, 0.0)
    out_ref[...] = (jnp.dot(a, w_ref[...], preferred_element_type=F32, precision=jax.lax.Precision.HIGHEST)
                    + b_ref[...]) * inv


def _tcmid(parts, hs, inv, w, b):
    return pl.pallas_call(
        _tcmid_body,
        out_shape=jax.ShapeDtypeStruct((N_PAD, w.shape[1]), F32),
    )(parts, hs, inv, w, b)


def _attention_row(x, wa_ref, n):
    ctx = jnp.tanh(jnp.sum(jnp.dot(x, wa_ref[...], preferred_element_type=F32, precision=jax.lax.Precision.HIGHEST),
                           axis=0, keepdims=True) / n)
    w = jax.nn.sigmoid(jnp.sum(x * ctx, axis=1, keepdims=True))
    return jnp.sum(x * w, axis=0, keepdims=True)


def _tail_body(p_ref, hs_ref, inv_ref, vx_ref, vei_ref,
               wv1_ref, bv1_ref, wv2_ref, bv2_ref, wv3_ref, bv3_ref,
               wa_ref, wntn_ref, vntnT_ref, bntn_ref,
               wfc_ref, bfc_ref, wp_ref, bp_ref, wval_ref, bval_ref,
               pi_ref, val_ref):
    p = p_ref[...]
    abs_sub = (p[0] + p[1] - hs_ref[...]) * inv_ref[...]
    row_ids = lax.broadcasted_iota(jnp.int32, (N_PAD, 1), 0)
    abs_sub = jnp.where(row_ids < N_SUB, abs_sub, 0.0)

    ei = vei_ref[...]
    src_row = ei[0:1, :]
    dst_row = ei[1:2, :]
    nodes = lax.broadcasted_iota(jnp.int32, (64, 256), 0)
    s_t = (nodes == src_row).astype(F32)
    d_t = (nodes == dst_row).astype(F32)
    cnt = jnp.sum(d_t, axis=1, keepdims=True)
    invv = lax.rsqrt(cnt + 1.0)
    adj = lax.dot_general(d_t, s_t, (((1,), (1,)), ((), ())),
                          preferred_element_type=F32, precision=jax.lax.Precision.HIGHEST)
    eye = (lax.broadcasted_iota(jnp.int32, (64, 64), 0)
           == lax.broadcasted_iota(jnp.int32, (64, 64), 1)).astype(F32)
    m = adj + eye

    def vagg(h):
        return invv * jnp.dot(m, h * invv, preferred_element_type=F32, precision=jax.lax.Precision.HIGHEST)

    vx = vx_ref[...]
    h1 = jnp.maximum(vagg(jnp.dot(vx, wv1_ref[...], preferred_element_type=F32, precision=jax.lax.Precision.HIGHEST)
                          + bv1_ref[...]), 0.0)
    h2 = jnp.maximum(vagg(jnp.dot(h1, wv2_ref[...], preferred_element_type=F32, precision=jax.lax.Precision.HIGHEST)
                          + bv2_ref[...]), 0.0)
    abs_vnr = vagg(jnp.dot(h2, wv3_ref[...], preferred_element_type=F32, precision=jax.lax.Precision.HIGHEST)
                   + bv3_ref[...])

    e1 = _attention_row(abs_sub, wa_ref, float(N_SUB))
    e2 = _attention_row(abs_vnr, wa_ref, 64.0)

    s = jnp.zeros((1, 16), F32)
    for i in range(16):
        s = s + e1[:, i:i + 1] * jnp.dot(e2, wntn_ref[i],
                                         preferred_element_type=F32, precision=jax.lax.Precision.HIGHEST)
    ecat = jnp.concatenate([e1, e2], axis=1)
    ntn = jnp.maximum(
        s + jnp.dot(ecat, vntnT_ref[...], preferred_element_type=F32, precision=jax.lax.Precision.HIGHEST)
        + bntn_ref[...], 0.0)

    final = jnp.maximum(
        jnp.dot(ntn, wfc_ref[...], preferred_element_type=F32, precision=jax.lax.Precision.HIGHEST) + bfc_ref[...],
        0.0)
    logits = jnp.dot(final, wp_ref[...], preferred_element_type=F32, precision=jax.lax.Precision.HIGHEST) + bp_ref[...]
    mx = jnp.max(logits, axis=1, keepdims=True)
    ex = jnp.exp(logits - mx)
    pi_ref[...] = ex / jnp.sum(ex, axis=1, keepdims=True)
    val_ref[...] = (jnp.dot(final, wval_ref[...], preferred_element_type=F32, precision=jax.lax.Precision.HIGHEST)
                    + bval_ref[...])


def _tail(*args):
    return pl.pallas_call(
        _tail_body,
        out_shape=[jax.ShapeDtypeStruct((1, N_SUB), F32),
                   jax.ShapeDtypeStruct((1, 1), F32)],
    )(*args)


def kernel(sub_x, sub_edge_index, vnr_x, vnr_edge_index,
           Ws1, bs1, Ws2, bs2, Ws3, bs3,
           Wv1, bv1, Wv2, bv2, Wv3, bv3,
           Wa, Wntn, Vntn, bntn, Wfc, bfc, Wp, bp, Wval, bval):
    sub_x_p = jnp.pad(sub_x, ((0, N_PAD - N_SUB), (0, 0)))
    ei = sub_edge_index.astype(jnp.int32)
    pad = E_PAD - E_SUB
    fill = jnp.full((pad,), N_SUB, jnp.int32)
    src = jnp.concatenate([ei[0], fill]).reshape(NW, K_CHUNKS, CHUNK)
    dst = jnp.concatenate([ei[1], fill]).reshape(NW, K_CHUNKS, CHUNK)
    ones = jnp.ones((N_PAD, DEG_W), F32)

    deg_p = _deg_call(ones, dst)
    hs1, inv = _tc1(deg_p, sub_x_p, Ws1, bs1.reshape(1, -1))
    p1 = _agg_64(hs1, src, dst)
    hs2 = _tcmid(p1, hs1, inv, Ws2, bs2.reshape(1, -1))
    p2 = _agg_32(hs2, src, dst)
    hs3 = _tcmid(p2, hs2, inv, Ws3, bs3.reshape(1, -1))
    p3 = _agg_16(hs3, src, dst)

    pi, value = _tail(
        p3, hs3, inv, vnr_x, vnr_edge_index.astype(jnp.int32),
        Wv1, bv1.reshape(1, -1), Wv2, bv2.reshape(1, -1),
        Wv3, bv3.reshape(1, -1),
        Wa, Wntn, Vntn.T, bntn.reshape(1, -1),
        Wfc, bfc.reshape(1, -1), Wp, bp.reshape(1, -1),
        Wval, bval.reshape(1, -1))
    return (pi, value)


# speedup vs baseline: 16.0978x; 1.3094x over previous
import functools

import jax
import jax.numpy as jnp
from jax import lax
from jax.experimental import pallas as pl
from jax.experimental.pallas import tpu as pltpu
from jax.experimental.pallas import tpu_sc as plsc

F32 = jnp.float32

N_SUB = 10000
N_PAD = 10112
E_SUB = 320000
NC, NS = 2, 16
NW = NC * NS
CHUNK = 128
NBUF = 4
K_CHUNKS = 80
E_PAD = NW * K_CHUNKS * CHUNK
ROWS_PER_TILE = N_PAD // NS
DEG_W = 16


def _sc_mesh():
    return plsc.VectorSubcoreMesh(
        core_axis_name="c", subcore_axis_name="s",
        num_cores=NC, num_subcores=NS)


def _deg_body(ones_hbm, dst_hbm, part_hbm, acc, ones_buf, didx_all, ssem):
    cid = lax.axis_index("c")
    sid = lax.axis_index("s")
    wid = cid * NS + sid
    r0 = sid * ROWS_PER_TILE
    pltpu.sync_copy(ones_hbm.at[pl.ds(r0, ROWS_PER_TILE)],
                    acc.at[pl.ds(r0, ROWS_PER_TILE)])
    pltpu.sync_copy(ones_hbm.at[pl.ds(0, CHUNK)], ones_buf)
    pltpu.sync_copy(dst_hbm.at[wid], didx_all)
    plsc.subcore_barrier()

    def fire(k, carry):
        pltpu.async_copy(ones_buf, acc.at[didx_all.at[k]], ssem, add=True)
        return carry

    lax.fori_loop(0, K_CHUNKS, fire, 0)

    def drain(k, carry):
        pltpu.make_async_copy(ones_buf, acc.at[didx_all.at[0]], ssem).wait()
        return carry

    lax.fori_loop(0, K_CHUNKS, drain, 0)
    plsc.subcore_barrier()
    pltpu.sync_copy(acc.at[pl.ds(r0, ROWS_PER_TILE)],
                    part_hbm.at[cid, pl.ds(r0, ROWS_PER_TILE)])


_sc_fns = {}


def _deg_call(ones, dst):
    if "deg" not in _sc_fns:
        _sc_fns["deg"] = functools.partial(
            pl.kernel,
            out_type=jax.ShapeDtypeStruct((NC, N_PAD, DEG_W), F32),
            mesh=_sc_mesh(),
            scratch_types=[
                pltpu.VMEM_SHARED((N_PAD, DEG_W), F32),
                pltpu.VMEM((CHUNK, DEG_W), F32),
                pltpu.VMEM((K_CHUNKS, CHUNK), jnp.int32),
                pltpu.SemaphoreType.DMA,
            ],
            compiler_params=pltpu.CompilerParams(use_tc_tiling_on_sc=False),
        )(_deg_body)
    return _sc_fns["deg"](ones, dst)


def _make_agg(feat):
    def body(hs_hbm, src_hbm, dst_hbm, part_hbm, acc,
             rows, sidx_all, didx_all, *gsems):
        cid = lax.axis_index("c")
        sid = lax.axis_index("s")
        wid = cid * NS + sid
        r0 = sid * ROWS_PER_TILE
        pltpu.sync_copy(hs_hbm.at[pl.ds(r0, ROWS_PER_TILE)],
                        acc.at[pl.ds(r0, ROWS_PER_TILE)])
        pltpu.sync_copy(src_hbm.at[wid], sidx_all)
        pltpu.sync_copy(dst_hbm.at[wid], didx_all)
        plsc.subcore_barrier()

        for b in range(NBUF):
            pltpu.async_copy(hs_hbm.at[sidx_all.at[b]], rows.at[b], gsems[b])

        def outer(i, carry):
            g = i * NBUF
            for b in range(NBUF):
                k = g + b
                pltpu.make_async_copy(
                    hs_hbm.at[sidx_all.at[b]], rows.at[b], gsems[b]).wait()
                pltpu.sync_copy(rows.at[b], acc.at[didx_all.at[k]], add=True)

                @pl.when(g < K_CHUNKS - NBUF)
                def _():
                    pltpu.async_copy(hs_hbm.at[sidx_all.at[k + NBUF]],
                                     rows.at[b], gsems[b])
            return carry

        lax.fori_loop(0, K_CHUNKS // NBUF, outer, 0)
        plsc.subcore_barrier()
        pltpu.sync_copy(acc.at[pl.ds(r0, ROWS_PER_TILE)],
                        part_hbm.at[cid, pl.ds(r0, ROWS_PER_TILE)])

    def call(hs, src, dst):
        key = ("agg", feat)
        if key not in _sc_fns:
            _sc_fns[key] = functools.partial(
                pl.kernel,
                out_type=jax.ShapeDtypeStruct((NC, N_PAD, feat), F32),
                mesh=_sc_mesh(),
                scratch_types=[
                    pltpu.VMEM_SHARED((N_PAD, feat), F32),
                    pltpu.VMEM((NBUF, CHUNK, feat), F32),
                    pltpu.VMEM((K_CHUNKS, CHUNK), jnp.int32),
                    pltpu.VMEM((K_CHUNKS, CHUNK), jnp.int32),
                ] + [pltpu.SemaphoreType.DMA] * NBUF,
                compiler_params=pltpu.CompilerParams(
                    use_tc_tiling_on_sc=False),
            )(body)
        return _sc_fns[key](hs, src, dst)

    return call


_agg_64 = _make_agg(64)
_agg_32 = _make_agg(32)
_agg_16 = _make_agg(16)


def _tc1_body(p_ref, x_ref, w_ref, b_ref, hs_ref, inv_ref):
    p = p_ref[...]
    deg = p[0, :, 0:1] + p[1, :, 0:1] - 1.0
    inv = lax.rsqrt(deg)
    h = jnp.dot(x_ref[...], w_ref[...], preferred_element_type=F32, precision=jax.lax.Precision.HIGHEST) + b_ref[...]
    hs_ref[...] = h * inv
    inv_ref[...] = inv


def _tc1(parts, x, w, b):
    return pl.pallas_call(
        _tc1_body,
        out_shape=[jax.ShapeDtypeStruct((N_PAD, w.shape[1]), F32),
                   jax.ShapeDtypeStruct((N_PAD, 1), F32)],
    )(parts, x, w, b)


def _tcmid_body(p_ref, hs_ref, inv_ref, w_ref, b_ref, out_ref):
    p = p_ref[...]
    inv = inv_ref[...]
    a = jnp.maximum((p[0] + p[1] - hs_ref[...]) * inv, 0.0)
    out_ref[...] = (jnp.dot(a, w_ref[...], preferred_element_type=F32, precision=jax.lax.Precision.HIGHEST)
                    + b_ref[...]) * inv


def _tcmid(parts, hs, inv, w, b):
    return pl.pallas_call(
        _tcmid_body,
        out_shape=jax.ShapeDtypeStruct((N_PAD, w.shape[1]), F32),
    )(parts, hs, inv, w, b)


def _attention_row(x, wa_ref, n):
    ctx = jnp.tanh(jnp.sum(jnp.dot(x, wa_ref[...], preferred_element_type=F32, precision=jax.lax.Precision.HIGHEST),
                           axis=0, keepdims=True) / n)
    w = jax.nn.sigmoid(jnp.sum(x * ctx, axis=1, keepdims=True))
    return jnp.sum(x * w, axis=0, keepdims=True)


def _tail_body(p_ref, hs_ref, inv_ref, vx_ref, vei_ref,
               wv1_ref, bv1_ref, wv2_ref, bv2_ref, wv3_ref, bv3_ref,
               wa_ref, wntn_ref, vntnT_ref, bntn_ref,
               wfc_ref, bfc_ref, wp_ref, bp_ref, wval_ref, bval_ref,
               pi_ref, val_ref):
    p = p_ref[...]
    abs_sub = (p[0] + p[1] - hs_ref[...]) * inv_ref[...]
    row_ids = lax.broadcasted_iota(jnp.int32, (N_PAD, 1), 0)
    abs_sub = jnp.where(row_ids < N_SUB, abs_sub, 0.0)

    ei = vei_ref[...]
    src_row = ei[0:1, :]
    dst_row = ei[1:2, :]
    nodes = lax.broadcasted_iota(jnp.int32, (64, 256), 0)
    s_t = (nodes == src_row).astype(F32)
    d_t = (nodes == dst_row).astype(F32)
    cnt = jnp.sum(d_t, axis=1, keepdims=True)
    invv = lax.rsqrt(cnt + 1.0)
    adj = lax.dot_general(d_t, s_t, (((1,), (1,)), ((), ())),
                          preferred_element_type=F32, precision=jax.lax.Precision.HIGHEST)
    eye = (lax.broadcasted_iota(jnp.int32, (64, 64), 0)
           == lax.broadcasted_iota(jnp.int32, (64, 64), 1)).astype(F32)
    m = adj + eye

    def vagg(h):
        return invv * jnp.dot(m, h * invv, preferred_element_type=F32, precision=jax.lax.Precision.HIGHEST)

    vx = vx_ref[...]
    h1 = jnp.maximum(vagg(jnp.dot(vx, wv1_ref[...], preferred_element_type=F32, precision=jax.lax.Precision.HIGHEST)
                          + bv1_ref[...]), 0.0)
    h2 = jnp.maximum(vagg(jnp.dot(h1, wv2_ref[...], preferred_element_type=F32, precision=jax.lax.Precision.HIGHEST)
                          + bv2_ref[...]), 0.0)
    abs_vnr = vagg(jnp.dot(h2, wv3_ref[...], preferred_element_type=F32, precision=jax.lax.Precision.HIGHEST)
                   + bv3_ref[...])

    e1 = _attention_row(abs_sub, wa_ref, float(N_SUB))
    e2 = _attention_row(abs_vnr, wa_ref, 64.0)

    s = jnp.zeros((1, 16), F32)
    for i in range(16):
        s = s + e1[:, i:i + 1] * jnp.dot(e2, wntn_ref[i],
                                         preferred_element_type=F32, precision=jax.lax.Precision.HIGHEST)
    ecat = jnp.concatenate([e1, e2], axis=1)
    ntn = jnp.maximum(
        s + jnp.dot(ecat, vntnT_ref[...], preferred_element_type=F32, precision=jax.lax.Precision.HIGHEST)
        + bntn_ref[...], 0.0)

    final = jnp.maximum(
        jnp.dot(ntn, wfc_ref[...], preferred_element_type=F32, precision=jax.lax.Precision.HIGHEST) + bfc_ref[...],
        0.0)
    logits = jnp.dot(final, wp_ref[...], preferred_element_type=F32, precision=jax.lax.Precision.HIGHEST) + bp_ref[...]
    mx = jnp.max(logits, axis=1, keepdims=True)
    ex = jnp.exp(logits - mx)
    pi_ref[...] = ex / jnp.sum(ex, axis=1, keepdims=True)
    val_ref[...] = (jnp.dot(final, wval_ref[...], preferred_element_type=F32, precision=jax.lax.Precision.HIGHEST)
                    + bval_ref[...])


def _tail(*args):
    return pl.pallas_call(
        _tail_body,
        out_shape=[jax.ShapeDtypeStruct((1, N_SUB), F32),
                   jax.ShapeDtypeStruct((1, 1), F32)],
    )(*args)


def kernel(sub_x, sub_edge_index, vnr_x, vnr_edge_index,
           Ws1, bs1, Ws2, bs2, Ws3, bs3,
           Wv1, bv1, Wv2, bv2, Wv3, bv3,
           Wa, Wntn, Vntn, bntn, Wfc, bfc, Wp, bp, Wval, bval):
    sub_x_p = jnp.pad(sub_x, ((0, N_PAD - N_SUB), (0, 0)))
    ei = sub_edge_index.astype(jnp.int32)
    pad = E_PAD - E_SUB
    fill = jnp.full((pad,), N_SUB, jnp.int32)
    src = jnp.concatenate([ei[0], fill]).reshape(NW, K_CHUNKS, CHUNK)
    dst = jnp.concatenate([ei[1], fill]).reshape(NW, K_CHUNKS, CHUNK)
    ones = jnp.ones((N_PAD, DEG_W), F32)

    deg_p = _deg_call(ones, dst)
    hs1, inv = _tc1(deg_p, sub_x_p, Ws1, bs1.reshape(1, -1))
    p1 = _agg_64(hs1, src, dst)
    hs2 = _tcmid(p1, hs1, inv, Ws2, bs2.reshape(1, -1))
    p2 = _agg_32(hs2, src, dst)
    hs3 = _tcmid(p2, hs2, inv, Ws3, bs3.reshape(1, -1))
    p3 = _agg_16(hs3, src, dst)

    pi, value = _tail(
        p3, hs3, inv, vnr_x, vnr_edge_index.astype(jnp.int32),
        Wv1, bv1.reshape(1, -1), Wv2, bv2.reshape(1, -1),
        Wv3, bv3.reshape(1, -1),
        Wa, Wntn, Vntn.T, bntn.reshape(1, -1),
        Wfc, bfc.reshape(1, -1), Wp, bp.reshape(1, -1),
        Wval, bval.reshape(1, -1))
    return (pi, value)
